# baseline (speedup 1.0000x reference)
import functools

import numpy as np

import jax
import jax.numpy as jnp
from jax import lax
from jax.experimental import pallas as pl
from jax.experimental.pallas import tpu as pltpu
from jax.experimental.pallas import tpu_sc as plsc

N = 10000
E = 320000
D = 128
HID = 64
C2 = 64

NC = 2
NS = 16
NW = NC * NS
EPW = E // NW
SUB = 125
NSUB = 2
CHUNK = SUB * NSUB
NCHUNK = EPW // CHUNK
ROWW = 80
MC = 64
ACCW = 72
NPAD = 10112
NPT = NPAD // NS
NROWZ = 79

_ROWBLK = 1000
_GRID = N // _ROWBLK


def _make_edge_kernel(nheads):
    mesh = plsc.VectorSubcoreMesh(core_axis_name="c", subcore_axis_name="s")

    @functools.partial(
        pl.kernel,
        out_type=jax.ShapeDtypeStruct((NC, NPAD, ACCW), jnp.float32),
        mesh=mesh,
        scratch_types=(
            pltpu.VMEM((NSUB, SUB), jnp.int32),
            pltpu.VMEM((NSUB, SUB), jnp.int32),
            pltpu.VMEM((NSUB, SUB), jnp.int32),
            pltpu.VMEM((NSUB, SUB), jnp.int32),
            pltpu.VMEM((CHUNK, ROWW), jnp.float32),
            pltpu.VMEM((CHUNK, ROWW), jnp.float32),
            pltpu.VMEM((CHUNK, 16), jnp.float32),
            pltpu.VMEM((CHUNK, 16), jnp.float32),
            pltpu.VMEM((CHUNK, ACCW), jnp.float32),
            pltpu.VMEM_SHARED((NPAD, ACCW), jnp.float32),
            pltpu.SemaphoreType.DMA,
            pltpu.SemaphoreType.DMA,
            pltpu.SemaphoreType.DMA,
            pltpu.SemaphoreType.DMA,
        ),
        compiler_params=pltpu.CompilerParams(use_tc_tiling_on_sc=False,
                                             needs_layout_passes=False),
    )
    def edge_kernel(htab, adtab, src_i, dst_i, acc_o,
                    srcv0, srcv1, dstv0, dstv1, g1a, g1b, g2a, g2b,
                    msgex, acc_s, s1a, s1b, s2a, s2b):
        c = lax.axis_index("c")
        s = lax.axis_index("s")
        wid = s * NC + c
        z16 = jnp.zeros((16,), jnp.float32)
        bufs = ((srcv0, dstv0, g1a, g2a, s1a, s2a),
                (srcv1, dstv1, g1b, g2b, s1b, s2b))

        def fire(ch, b):
            si, di, g1, g2, sh, sa = bufs[b]
            rowbase = wid * (EPW // SUB) + ch * NSUB
            pltpu.sync_copy(src_i.at[pl.ds(rowbase, NSUB)], si)
            pltpu.sync_copy(dst_i.at[pl.ds(rowbase, NSUB)], di)
            for j in range(NSUB):
                pltpu.async_copy(htab.at[si.at[j]],
                                 g1.at[pl.ds(j * SUB, SUB)], sh)
                pltpu.async_copy(adtab.at[di.at[j]],
                                 g2.at[pl.ds(j * SUB, SUB)], sa)

        def drain(b):
            si, di, g1, g2, sh, sa = bufs[b]
            for j in range(NSUB):
                pltpu.make_async_copy(htab.at[si.at[j]],
                                      g1.at[pl.ds(j * SUB, SUB)], sh).wait()
                pltpu.make_async_copy(adtab.at[di.at[j]],
                                      g2.at[pl.ds(j * SUB, SUB)], sa).wait()

        def zrow(r, carry):
            for k in (0, 16, 32, 48, 56):
                msgex[r, pl.ds(k, 16)] = z16
            return carry

        lax.fori_loop(0, NROWZ, zrow, 0)
        for t in range(NPT // NROWZ):
            pltpu.sync_copy(msgex.at[pl.ds(0, NROWZ)],
                            acc_s.at[pl.ds(s * NPT + t * NROWZ, NROWZ)])
        plsc.subcore_barrier()

        gdn = lax.GatherDimensionNumbers(
            offset_dims=(), collapsed_slice_dims=(0,), start_index_map=(0,))

        def compute_scatter(b):
            si, di, g1, g2, sh, sa = bufs[b]

            @plsc.parallel_loop(0, CHUNK, unroll=8)
            def edge_body(e):
                av = g1[e, pl.ds(MC, 16)]
                bv = g2[e, pl.ds(0, 16)]
                es = av + bv
                es = jnp.maximum(es, es * jnp.float32(0.2))
                ex = jnp.exp(es)
                erow = jnp.full((16,), e, jnp.int32)
                iotl = lax.iota(jnp.int32, 16)
                plsc.store_scatter(msgex, [erow, MC + (iotl % 8)], ex,
                                   mask=iotl < 8)
                for v in range(MC // 16):
                    if nheads == 8:
                        pv = 2 * v + (iotl // 8)
                    else:
                        pv = 0 * (iotl // 8)
                    hx = g1[e, pl.ds(16 * v, 16)]
                    exv = lax.gather(
                        ex, pv[:, None], gdn, (1,),
                        mode=lax.GatherScatterMode.PROMISE_IN_BOUNDS)
                    msgex[e, pl.ds(16 * v, 16)] = hx * exv
            for j in range(NSUB):
                pltpu.sync_copy(msgex.at[pl.ds(j * SUB, SUB)],
                                acc_s.at[di.at[j]], add=True)

        fire(0, 0)

        def pair_body(i, carry):
            fire(2 * i + 1, 1)
            drain(0)
            compute_scatter(0)

            @pl.when(i < NCHUNK // 2 - 1)
            def _():
                fire(2 * i + 2, 0)

            drain(1)
            compute_scatter(1)
            return carry

        lax.fori_loop(0, NCHUNK // 2, pair_body, 0)
        plsc.subcore_barrier()

        pltpu.sync_copy(acc_s.at[pl.ds(s * NPT, NPT)],
                        acc_o.at[c, pl.ds(s * NPT, NPT)])

    return edge_kernel


_edge_kernel_l1 = _make_edge_kernel(8)
_edge_kernel_l2 = _make_edge_kernel(1)


def _tc1_body(x_ref, w1_ref, asd_ref, ad_ref, htab_ref, adtab_ref):
    h = jnp.dot(x_ref[...], w1_ref[...], preferred_element_type=jnp.float32)
    sa = jnp.dot(h, asd_ref[...], preferred_element_type=jnp.float32)
    htab_ref[...] = jnp.concatenate([h, sa], axis=1)
    adtab_ref[...] = jnp.dot(h, ad_ref[...], preferred_element_type=jnp.float32)


def _tc2_body(acc_ref, erep_ref, b1_ref, w2_ref, a2sd_ref, a2d_ref,
              htab_ref, adtab_ref):
    both = acc_ref[0] + acc_ref[1]
    acc = both[:, :MC]
    den = both[:, MC:ACCW]
    dex = jnp.dot(den, erep_ref[...], preferred_element_type=jnp.float32)
    h1 = acc / (dex + 1e-16) + b1_ref[...]
    h1 = jnp.where(h1 > 0, h1, jnp.exp(jnp.minimum(h1, 0.0)) - 1.0)
    h2 = jnp.dot(h1, w2_ref[...], preferred_element_type=jnp.float32)
    sa = jnp.dot(h2, a2sd_ref[...], preferred_element_type=jnp.float32)
    htab_ref[...] = jnp.concatenate([h2, sa], axis=1)
    adtab_ref[...] = jnp.dot(h2, a2d_ref[...], preferred_element_type=jnp.float32)


def _tc3_body(acc_ref, bmat_ref, b2_ref, out_ref):
    both = acc_ref[0] + acc_ref[1]
    acc = both[:, :MC]
    den = both[:, MC:ACCW]
    dex = jnp.dot(den, bmat_ref[...], preferred_element_type=jnp.float32)
    v = acc / (dex + 1e-16) + b2_ref[...]
    m = jnp.max(v, axis=1, keepdims=True)
    z = v - m
    out_ref[...] = z - jnp.log(jnp.sum(jnp.exp(z), axis=1, keepdims=True))


def _full(shape):
    return pl.BlockSpec(shape, lambda i: tuple(0 for _ in shape))


def kernel(x, edge_index, W1, a_src1, a_dst1, b1, W2, a_src2, a_dst2, b2):
    f32 = jnp.float32
    src2d = edge_index[0].reshape(E // SUB, SUB)
    dst2d = edge_index[1].reshape(E // SUB, SUB)

    eye8 = jnp.eye(8, dtype=f32)
    as64 = (a_src1[:, :, None] * eye8[:, None, :]).reshape(64, 8)
    ad64 = (a_dst1[:, :, None] * eye8[:, None, :]).reshape(64, 8)
    asd = jnp.concatenate([as64, ad64], axis=1)
    ad16 = jnp.concatenate([ad64, jnp.zeros((64, 8), f32)], axis=1)
    erep = jnp.repeat(jnp.eye(8, dtype=f32), 8, axis=1)
    a2sd = jnp.concatenate([a_src2.T, jnp.zeros((64, 15), f32)], axis=1)
    a2d = jnp.concatenate([a_dst2.T, jnp.zeros((64, 15), f32)], axis=1)
    bmat = jnp.concatenate([jnp.ones((1, 64), f32),
                            jnp.zeros((7, 64), f32)], axis=0)
    b1r = b1.reshape(1, HID)
    b2r = b2.reshape(1, C2)

    htab1, adtab1 = pl.pallas_call(
        _tc1_body,
        grid=(_GRID,),
        in_specs=[
            pl.BlockSpec((_ROWBLK, D), lambda i: (i, 0)),
            _full((D, HID)),
            _full((64, 16)),
            _full((64, 16)),
        ],
        out_specs=[
            pl.BlockSpec((_ROWBLK, ROWW), lambda i: (i, 0)),
            pl.BlockSpec((_ROWBLK, 16), lambda i: (i, 0)),
        ],
        out_shape=[
            jax.ShapeDtypeStruct((N, ROWW), f32),
            jax.ShapeDtypeStruct((N, 16), f32),
        ],
    )(x, W1, asd, ad16)

    acc1 = _edge_kernel_l1(htab1, adtab1, src2d, dst2d)

    htab2, adtab2 = pl.pallas_call(
        _tc2_body,
        grid=(_GRID,),
        in_specs=[
            pl.BlockSpec((NC, _ROWBLK, ACCW), lambda i: (0, i, 0)),
            _full((8, 64)),
            _full((1, HID)),
            _full((HID, C2)),
            _full((64, 16)),
            _full((64, 16)),
        ],
        out_specs=[
            pl.BlockSpec((_ROWBLK, ROWW), lambda i: (i, 0)),
            pl.BlockSpec((_ROWBLK, 16), lambda i: (i, 0)),
        ],
        out_shape=[
            jax.ShapeDtypeStruct((N, ROWW), f32),
            jax.ShapeDtypeStruct((N, 16), f32),
        ],
    )(acc1, erep, b1r, W2, a2sd, a2d)

    acc2 = _edge_kernel_l2(htab2, adtab2, src2d, dst2d)

    out = pl.pallas_call(
        _tc3_body,
        grid=(_GRID,),
        in_specs=[
            pl.BlockSpec((NC, _ROWBLK, ACCW), lambda i: (0, i, 0)),
            _full((8, 64)),
            _full((1, C2)),
        ],
        out_specs=pl.BlockSpec((_ROWBLK, C2), lambda i: (i, 0)),
        out_shape=jax.ShapeDtypeStruct((N, C2), f32),
    )(acc2, bmat, b2r)

    return out


# speedup vs baseline: 1.0022x; 1.0022x over previous
import functools

import numpy as np

import jax
import jax.numpy as jnp
from jax import lax
from jax.experimental import pallas as pl
from jax.experimental.pallas import tpu as pltpu
from jax.experimental.pallas import tpu_sc as plsc

N = 10000
E = 320000
D = 128
HID = 64
C2 = 64

NC = 2
NS = 16
NW = NC * NS
EPW = E // NW
SUB = 125
NSUB = 2
CHUNK = SUB * NSUB
NCHUNK = EPW // CHUNK
ROWW = 80
MC = 64
ACCW = 72
NPAD = 10112
NPT = NPAD // NS
NROWZ = 79

_ROWBLK = 1000
_GRID = N // _ROWBLK


def _make_edge_kernel(nheads):
    mesh = plsc.VectorSubcoreMesh(core_axis_name="c", subcore_axis_name="s")

    @functools.partial(
        pl.kernel,
        out_type=jax.ShapeDtypeStruct((NC, NPAD, ACCW), jnp.float32),
        mesh=mesh,
        scratch_types=(
            pltpu.VMEM((NSUB, SUB), jnp.int32),
            pltpu.VMEM((NSUB, SUB), jnp.int32),
            pltpu.VMEM((NSUB, SUB), jnp.int32),
            pltpu.VMEM((NSUB, SUB), jnp.int32),
            pltpu.VMEM((CHUNK, ROWW), jnp.float32),
            pltpu.VMEM((CHUNK, ROWW), jnp.float32),
            pltpu.VMEM((CHUNK, 16), jnp.float32),
            pltpu.VMEM((CHUNK, 16), jnp.float32),
            pltpu.VMEM((CHUNK, ACCW), jnp.float32),
            pltpu.VMEM_SHARED((NPAD, ACCW), jnp.float32),
            pltpu.SemaphoreType.DMA,
            pltpu.SemaphoreType.DMA,
            pltpu.SemaphoreType.DMA,
            pltpu.SemaphoreType.DMA,
        ),
        compiler_params=pltpu.CompilerParams(use_tc_tiling_on_sc=False,
                                             needs_layout_passes=False),
    )
    def edge_kernel(htab, adtab, src_i, dst_i, acc_o,
                    srcv0, srcv1, dstv0, dstv1, g1a, g1b, g2a, g2b,
                    msgex, acc_s, s1a, s1b, s2a, s2b):
        c = lax.axis_index("c")
        s = lax.axis_index("s")
        wid = s * NC + c
        z16 = jnp.zeros((16,), jnp.float32)
        bufs = ((srcv0, dstv0, g1a, g2a, s1a, s2a),
                (srcv1, dstv1, g1b, g2b, s1b, s2b))

        def fire(ch, b):
            si, di, g1, g2, sh, sa = bufs[b]
            rowbase = wid * (EPW // SUB) + ch * NSUB
            pltpu.sync_copy(src_i.at[pl.ds(rowbase, NSUB)], si)
            pltpu.sync_copy(dst_i.at[pl.ds(rowbase, NSUB)], di)
            for j in range(NSUB):
                pltpu.async_copy(htab.at[si.at[j]],
                                 g1.at[pl.ds(j * SUB, SUB)], sh)
                pltpu.async_copy(adtab.at[di.at[j]],
                                 g2.at[pl.ds(j * SUB, SUB)], sa)

        def drain(b):
            si, di, g1, g2, sh, sa = bufs[b]
            for j in range(NSUB):
                pltpu.make_async_copy(htab.at[si.at[j]],
                                      g1.at[pl.ds(j * SUB, SUB)], sh).wait()
                pltpu.make_async_copy(adtab.at[di.at[j]],
                                      g2.at[pl.ds(j * SUB, SUB)], sa).wait()

        def zrow(r, carry):
            for k in (0, 16, 32, 48, 56):
                msgex[r, pl.ds(k, 16)] = z16
            return carry

        lax.fori_loop(0, NROWZ, zrow, 0)
        for t in range(NPT // NROWZ):
            pltpu.sync_copy(msgex.at[pl.ds(0, NROWZ)],
                            acc_s.at[pl.ds(s * NPT + t * NROWZ, NROWZ)])
        plsc.subcore_barrier()

        gdn = lax.GatherDimensionNumbers(
            offset_dims=(), collapsed_slice_dims=(0,), start_index_map=(0,))

        def compute_scatter(b):
            si, di, g1, g2, sh, sa = bufs[b]

            @plsc.parallel_loop(0, CHUNK, unroll=4)
            def edge_body(e):
                av = g1[e, pl.ds(MC, 16)]
                bv = g2[e, pl.ds(0, 16)]
                es = av + bv
                es = jnp.maximum(es, es * jnp.float32(0.2))
                ex = jnp.exp(es)
                erow = jnp.full((16,), e, jnp.int32)
                iotl = lax.iota(jnp.int32, 16)
                plsc.store_scatter(msgex, [erow, MC + (iotl % 8)], ex,
                                   mask=iotl < 8)
                for v in range(MC // 16):
                    if nheads == 8:
                        pv = 2 * v + (iotl // 8)
                    else:
                        pv = 0 * (iotl // 8)
                    hx = g1[e, pl.ds(16 * v, 16)]
                    exv = lax.gather(
                        ex, pv[:, None], gdn, (1,),
                        mode=lax.GatherScatterMode.PROMISE_IN_BOUNDS)
                    msgex[e, pl.ds(16 * v, 16)] = hx * exv
            for j in range(NSUB):
                pltpu.sync_copy(msgex.at[pl.ds(j * SUB, SUB)],
                                acc_s.at[di.at[j]], add=True)

        fire(0, 0)

        def pair_body(i, carry):
            fire(2 * i + 1, 1)
            drain(0)
            compute_scatter(0)

            @pl.when(i < NCHUNK // 2 - 1)
            def _():
                fire(2 * i + 2, 0)

            drain(1)
            compute_scatter(1)
            return carry

        lax.fori_loop(0, NCHUNK // 2, pair_body, 0)
        plsc.subcore_barrier()

        pltpu.sync_copy(acc_s.at[pl.ds(s * NPT, NPT)],
                        acc_o.at[c, pl.ds(s * NPT, NPT)])

    return edge_kernel


_edge_kernel_l1 = _make_edge_kernel(8)
_edge_kernel_l2 = _make_edge_kernel(1)


def _tc1_body(x_ref, w1_ref, asd_ref, ad_ref, htab_ref, adtab_ref):
    h = jnp.dot(x_ref[...], w1_ref[...], preferred_element_type=jnp.float32)
    sa = jnp.dot(h, asd_ref[...], preferred_element_type=jnp.float32)
    htab_ref[...] = jnp.concatenate([h, sa], axis=1)
    adtab_ref[...] = jnp.dot(h, ad_ref[...], preferred_element_type=jnp.float32)


def _tc2_body(acc_ref, erep_ref, b1_ref, w2_ref, a2sd_ref, a2d_ref,
              htab_ref, adtab_ref):
    both = acc_ref[0] + acc_ref[1]
    acc = both[:, :MC]
    den = both[:, MC:ACCW]
    dex = jnp.dot(den, erep_ref[...], preferred_element_type=jnp.float32)
    h1 = acc / (dex + 1e-16) + b1_ref[...]
    h1 = jnp.where(h1 > 0, h1, jnp.exp(jnp.minimum(h1, 0.0)) - 1.0)
    h2 = jnp.dot(h1, w2_ref[...], preferred_element_type=jnp.float32)
    sa = jnp.dot(h2, a2sd_ref[...], preferred_element_type=jnp.float32)
    htab_ref[...] = jnp.concatenate([h2, sa], axis=1)
    adtab_ref[...] = jnp.dot(h2, a2d_ref[...], preferred_element_type=jnp.float32)


def _tc3_body(acc_ref, bmat_ref, b2_ref, out_ref):
    both = acc_ref[0] + acc_ref[1]
    acc = both[:, :MC]
    den = both[:, MC:ACCW]
    dex = jnp.dot(den, bmat_ref[...], preferred_element_type=jnp.float32)
    v = acc / (dex + 1e-16) + b2_ref[...]
    m = jnp.max(v, axis=1, keepdims=True)
    z = v - m
    out_ref[...] = z - jnp.log(jnp.sum(jnp.exp(z), axis=1, keepdims=True))


def _full(shape):
    return pl.BlockSpec(shape, lambda i: tuple(0 for _ in shape))


def kernel(x, edge_index, W1, a_src1, a_dst1, b1, W2, a_src2, a_dst2, b2):
    f32 = jnp.float32
    src2d = edge_index[0].reshape(E // SUB, SUB)
    dst2d = edge_index[1].reshape(E // SUB, SUB)

    eye8 = jnp.eye(8, dtype=f32)
    as64 = (a_src1[:, :, None] * eye8[:, None, :]).reshape(64, 8)
    ad64 = (a_dst1[:, :, None] * eye8[:, None, :]).reshape(64, 8)
    asd = jnp.concatenate([as64, ad64], axis=1)
    ad16 = jnp.concatenate([ad64, jnp.zeros((64, 8), f32)], axis=1)
    erep = jnp.repeat(jnp.eye(8, dtype=f32), 8, axis=1)
    a2sd = jnp.concatenate([a_src2.T, jnp.zeros((64, 15), f32)], axis=1)
    a2d = jnp.concatenate([a_dst2.T, jnp.zeros((64, 15), f32)], axis=1)
    bmat = jnp.concatenate([jnp.ones((1, 64), f32),
                            jnp.zeros((7, 64), f32)], axis=0)
    b1r = b1.reshape(1, HID)
    b2r = b2.reshape(1, C2)

    htab1, adtab1 = pl.pallas_call(
        _tc1_body,
        grid=(_GRID,),
        in_specs=[
            pl.BlockSpec((_ROWBLK, D), lambda i: (i, 0)),
            _full((D, HID)),
            _full((64, 16)),
            _full((64, 16)),
        ],
        out_specs=[
            pl.BlockSpec((_ROWBLK, ROWW), lambda i: (i, 0)),
            pl.BlockSpec((_ROWBLK, 16), lambda i: (i, 0)),
        ],
        out_shape=[
            jax.ShapeDtypeStruct((N, ROWW), f32),
            jax.ShapeDtypeStruct((N, 16), f32),
        ],
    )(x, W1, asd, ad16)

    acc1 = _edge_kernel_l1(htab1, adtab1, src2d, dst2d)

    htab2, adtab2 = pl.pallas_call(
        _tc2_body,
        grid=(_GRID,),
        in_specs=[
            pl.BlockSpec((NC, _ROWBLK, ACCW), lambda i: (0, i, 0)),
            _full((8, 64)),
            _full((1, HID)),
            _full((HID, C2)),
            _full((64, 16)),
            _full((64, 16)),
        ],
        out_specs=[
            pl.BlockSpec((_ROWBLK, ROWW), lambda i: (i, 0)),
            pl.BlockSpec((_ROWBLK, 16), lambda i: (i, 0)),
        ],
        out_shape=[
            jax.ShapeDtypeStruct((N, ROWW), f32),
            jax.ShapeDtypeStruct((N, 16), f32),
        ],
    )(acc1, erep, b1r, W2, a2sd, a2d)

    acc2 = _edge_kernel_l2(htab2, adtab2, src2d, dst2d)

    out = pl.pallas_call(
        _tc3_body,
        grid=(_GRID,),
        in_specs=[
            pl.BlockSpec((NC, _ROWBLK, ACCW), lambda i: (0, i, 0)),
            _full((8, 64)),
            _full((1, C2)),
        ],
        out_specs=pl.BlockSpec((_ROWBLK, C2), lambda i: (i, 0)),
        out_shape=jax.ShapeDtypeStruct((N, C2), f32),
    )(acc2, bmat, b2r)

    return out


# speedup vs baseline: 1.1534x; 1.1509x over previous
import functools

import numpy as np

import jax
import jax.numpy as jnp
from jax import lax
from jax.experimental import pallas as pl
from jax.experimental.pallas import tpu as pltpu
from jax.experimental.pallas import tpu_sc as plsc

N = 10000
E = 320000
D = 128
HID = 64
C2 = 64

NC = 2
NS = 16
NW = NC * NS
EPW = E // NW
SUB = 125
NSUB = 2
CHUNK = SUB * NSUB
NCHUNK = EPW // CHUNK
ROWW = 80
MC = 64
ACCW = 72
NPAD = 10112
NPT = NPAD // NS
NROWZ = 79

_ROWBLK = 1000
_GRID = N // _ROWBLK


def _make_edge_kernel(nheads):
    mesh = plsc.VectorSubcoreMesh(core_axis_name="c", subcore_axis_name="s")

    @functools.partial(
        pl.kernel,
        out_type=jax.ShapeDtypeStruct((NC, NPAD, ACCW), jnp.float32),
        mesh=mesh,
        scratch_types=(
            pltpu.VMEM((NSUB, SUB), jnp.int32),
            pltpu.VMEM((NSUB, SUB), jnp.int32),
            pltpu.VMEM((NSUB, SUB), jnp.int32),
            pltpu.VMEM((NSUB, SUB), jnp.int32),
            pltpu.VMEM((CHUNK, ROWW), jnp.float32),
            pltpu.VMEM((CHUNK, ROWW), jnp.float32),
            pltpu.VMEM((CHUNK, 16), jnp.float32),
            pltpu.VMEM((CHUNK, 16), jnp.float32),
            pltpu.VMEM((CHUNK, ACCW), jnp.float32),
            pltpu.VMEM((CHUNK, ACCW), jnp.float32),
            pltpu.VMEM((NSUB, SUB), jnp.int32),
            pltpu.VMEM((NSUB, SUB), jnp.int32),
            pltpu.VMEM_SHARED((NPAD, ACCW), jnp.float32),
            pltpu.SemaphoreType.DMA,
            pltpu.SemaphoreType.DMA,
            pltpu.SemaphoreType.DMA,
            pltpu.SemaphoreType.DMA,
            pltpu.SemaphoreType.DMA,
            pltpu.SemaphoreType.DMA,
        ),
        compiler_params=pltpu.CompilerParams(use_tc_tiling_on_sc=False,
                                             needs_layout_passes=False),
    )
    def edge_kernel(htab, adtab, src_i, dst_i, acc_o,
                    srcv0, srcv1, dstv0, dstv1, g1a, g1b, g2a, g2b,
                    mxa, mxb, dsc0, dsc1, acc_s, s1a, s1b, s2a, s2b,
                    ssa, ssb):
        c = lax.axis_index("c")
        s = lax.axis_index("s")
        wid = s * NC + c
        z16 = jnp.zeros((16,), jnp.float32)
        bufs = ((srcv0, dstv0, g1a, g2a, s1a, s2a, mxa, dsc0, ssa),
                (srcv1, dstv1, g1b, g2b, s1b, s2b, mxb, dsc1, ssb))

        def fire(ch, b):
            si, di, g1, g2, sh, sa, mx, dsc, ss = bufs[b]
            rowbase = wid * (EPW // SUB) + ch * NSUB
            pltpu.sync_copy(src_i.at[pl.ds(rowbase, NSUB)], si)
            pltpu.sync_copy(dst_i.at[pl.ds(rowbase, NSUB)], di)
            for j in range(NSUB):
                pltpu.async_copy(htab.at[si.at[j]],
                                 g1.at[pl.ds(j * SUB, SUB)], sh)
                pltpu.async_copy(adtab.at[di.at[j]],
                                 g2.at[pl.ds(j * SUB, SUB)], sa)

        def drain(b):
            si, di, g1, g2, sh, sa, mx, dsc, ss = bufs[b]
            for j in range(NSUB):
                pltpu.make_async_copy(htab.at[si.at[j]],
                                      g1.at[pl.ds(j * SUB, SUB)], sh).wait()
                pltpu.make_async_copy(adtab.at[di.at[j]],
                                      g2.at[pl.ds(j * SUB, SUB)], sa).wait()

        def zrow(r, carry):
            for k in (0, 16, 32, 48, 56):
                mxa[r, pl.ds(k, 16)] = z16
            return carry

        lax.fori_loop(0, NROWZ, zrow, 0)
        for t in range(NPT // NROWZ):
            pltpu.sync_copy(mxa.at[pl.ds(0, NROWZ)],
                            acc_s.at[pl.ds(s * NPT + t * NROWZ, NROWZ)])
        plsc.subcore_barrier()

        gdn = lax.GatherDimensionNumbers(
            offset_dims=(), collapsed_slice_dims=(0,), start_index_map=(0,))

        def compute_scatter(b, i):
            si, di, g1, g2, sh, sa, mx, dsc, ss = bufs[b]

            @pl.when(i > 0)
            def _():
                for j in range(NSUB):
                    pltpu.make_async_copy(
                        mx.at[pl.ds(j * SUB, SUB)],
                        acc_s.at[dsc.at[j]], ss).wait()

            @plsc.parallel_loop(0, CHUNK, unroll=4)
            def edge_body(e):
                av = g1[e, pl.ds(MC, 16)]
                bv = g2[e, pl.ds(0, 16)]
                es = av + bv
                es = jnp.maximum(es, es * jnp.float32(0.2))
                ex = jnp.exp(es)
                erow = jnp.full((16,), e, jnp.int32)
                iotl = lax.iota(jnp.int32, 16)
                plsc.store_scatter(mx, [erow, MC + (iotl % 8)], ex,
                                   mask=iotl < 8)
                for v in range(MC // 16):
                    if nheads == 8:
                        pv = 2 * v + (iotl // 8)
                    else:
                        pv = 0 * (iotl // 8)
                    hx = g1[e, pl.ds(16 * v, 16)]
                    exv = lax.gather(
                        ex, pv[:, None], gdn, (1,),
                        mode=lax.GatherScatterMode.PROMISE_IN_BOUNDS)
                    mx[e, pl.ds(16 * v, 16)] = hx * exv
            for j in range(NSUB):
                for o in (0, 16, 32, 48, 64, 80, 96, SUB - 16):
                    dsc[j, pl.ds(o, 16)] = di[j, pl.ds(o, 16)]
            for j in range(NSUB):
                pltpu.async_copy(mx.at[pl.ds(j * SUB, SUB)],
                                 acc_s.at[dsc.at[j]], ss, add=True)

        fire(0, 0)

        def pair_body(i, carry):
            fire(2 * i + 1, 1)
            drain(0)
            compute_scatter(0, i)

            @pl.when(i < NCHUNK // 2 - 1)
            def _():
                fire(2 * i + 2, 0)

            drain(1)
            compute_scatter(1, i)
            return carry

        lax.fori_loop(0, NCHUNK // 2, pair_body, 0)
        for b in range(2):
            si, di, g1, g2, sh, sa, mx, dsc, ss = bufs[b]
            for j in range(NSUB):
                pltpu.make_async_copy(mx.at[pl.ds(j * SUB, SUB)],
                                      acc_s.at[dsc.at[j]], ss).wait()
        plsc.subcore_barrier()

        pltpu.sync_copy(acc_s.at[pl.ds(s * NPT, NPT)],
                        acc_o.at[c, pl.ds(s * NPT, NPT)])

    return edge_kernel


_edge_kernel_l1 = _make_edge_kernel(8)
_edge_kernel_l2 = _make_edge_kernel(1)


def _tc1_body(x_ref, w1_ref, asd_ref, ad_ref, htab_ref, adtab_ref):
    h = jnp.dot(x_ref[...], w1_ref[...], preferred_element_type=jnp.float32)
    sa = jnp.dot(h, asd_ref[...], preferred_element_type=jnp.float32)
    htab_ref[...] = jnp.concatenate([h, sa], axis=1)
    adtab_ref[...] = jnp.dot(h, ad_ref[...], preferred_element_type=jnp.float32)


def _tc2_body(acc_ref, erep_ref, b1_ref, w2_ref, a2sd_ref, a2d_ref,
              htab_ref, adtab_ref):
    both = acc_ref[0] + acc_ref[1]
    acc = both[:, :MC]
    den = both[:, MC:ACCW]
    dex = jnp.dot(den, erep_ref[...], preferred_element_type=jnp.float32)
    h1 = acc / (dex + 1e-16) + b1_ref[...]
    h1 = jnp.where(h1 > 0, h1, jnp.exp(jnp.minimum(h1, 0.0)) - 1.0)
    h2 = jnp.dot(h1, w2_ref[...], preferred_element_type=jnp.float32)
    sa = jnp.dot(h2, a2sd_ref[...], preferred_element_type=jnp.float32)
    htab_ref[...] = jnp.concatenate([h2, sa], axis=1)
    adtab_ref[...] = jnp.dot(h2, a2d_ref[...], preferred_element_type=jnp.float32)


def _tc3_body(acc_ref, bmat_ref, b2_ref, out_ref):
    both = acc_ref[0] + acc_ref[1]
    acc = both[:, :MC]
    den = both[:, MC:ACCW]
    dex = jnp.dot(den, bmat_ref[...], preferred_element_type=jnp.float32)
    v = acc / (dex + 1e-16) + b2_ref[...]
    m = jnp.max(v, axis=1, keepdims=True)
    z = v - m
    out_ref[...] = z - jnp.log(jnp.sum(jnp.exp(z), axis=1, keepdims=True))


def _full(shape):
    return pl.BlockSpec(shape, lambda i: tuple(0 for _ in shape))


def kernel(x, edge_index, W1, a_src1, a_dst1, b1, W2, a_src2, a_dst2, b2):
    f32 = jnp.float32
    src2d = edge_index[0].reshape(E // SUB, SUB)
    dst2d = edge_index[1].reshape(E // SUB, SUB)

    eye8 = jnp.eye(8, dtype=f32)
    as64 = (a_src1[:, :, None] * eye8[:, None, :]).reshape(64, 8)
    ad64 = (a_dst1[:, :, None] * eye8[:, None, :]).reshape(64, 8)
    asd = jnp.concatenate([as64, ad64], axis=1)
    ad16 = jnp.concatenate([ad64, jnp.zeros((64, 8), f32)], axis=1)
    erep = jnp.repeat(jnp.eye(8, dtype=f32), 8, axis=1)
    a2sd = jnp.concatenate([a_src2.T, jnp.zeros((64, 15), f32)], axis=1)
    a2d = jnp.concatenate([a_dst2.T, jnp.zeros((64, 15), f32)], axis=1)
    bmat = jnp.concatenate([jnp.ones((1, 64), f32),
                            jnp.zeros((7, 64), f32)], axis=0)
    b1r = b1.reshape(1, HID)
    b2r = b2.reshape(1, C2)

    htab1, adtab1 = pl.pallas_call(
        _tc1_body,
        grid=(_GRID,),
        in_specs=[
            pl.BlockSpec((_ROWBLK, D), lambda i: (i, 0)),
            _full((D, HID)),
            _full((64, 16)),
            _full((64, 16)),
        ],
        out_specs=[
            pl.BlockSpec((_ROWBLK, ROWW), lambda i: (i, 0)),
            pl.BlockSpec((_ROWBLK, 16), lambda i: (i, 0)),
        ],
        out_shape=[
            jax.ShapeDtypeStruct((N, ROWW), f32),
            jax.ShapeDtypeStruct((N, 16), f32),
        ],
    )(x, W1, asd, ad16)

    acc1 = _edge_kernel_l1(htab1, adtab1, src2d, dst2d)

    htab2, adtab2 = pl.pallas_call(
        _tc2_body,
        grid=(_GRID,),
        in_specs=[
            pl.BlockSpec((NC, _ROWBLK, ACCW), lambda i: (0, i, 0)),
            _full((8, 64)),
            _full((1, HID)),
            _full((HID, C2)),
            _full((64, 16)),
            _full((64, 16)),
        ],
        out_specs=[
            pl.BlockSpec((_ROWBLK, ROWW), lambda i: (i, 0)),
            pl.BlockSpec((_ROWBLK, 16), lambda i: (i, 0)),
        ],
        out_shape=[
            jax.ShapeDtypeStruct((N, ROWW), f32),
            jax.ShapeDtypeStruct((N, 16), f32),
        ],
    )(acc1, erep, b1r, W2, a2sd, a2d)

    acc2 = _edge_kernel_l2(htab2, adtab2, src2d, dst2d)

    out = pl.pallas_call(
        _tc3_body,
        grid=(_GRID,),
        in_specs=[
            pl.BlockSpec((NC, _ROWBLK, ACCW), lambda i: (0, i, 0)),
            _full((8, 64)),
            _full((1, C2)),
        ],
        out_specs=pl.BlockSpec((_ROWBLK, C2), lambda i: (i, 0)),
        out_shape=jax.ShapeDtypeStruct((N, C2), f32),
    )(acc2, bmat, b2r)

    return out
